# baseline (speedup 1.0000x reference)
import functools

import jax
import jax.numpy as jnp
import numpy as np
from jax import lax
from jax.experimental import pallas as pl
from jax.experimental.pallas import tpu as pltpu
from jax.experimental.pallas import tpu_sc as plsc

SEQ_LENGTH = 8192
EMBEDDING_DIM = 64
NUM_CORES = 2
NUM_SUBCORES = 16
NUM_WORKERS = NUM_CORES * NUM_SUBCORES
ROWS_PER_WORKER = SEQ_LENGTH // NUM_WORKERS
GATHER_CHUNK = 128
CHUNKS_PER_WORKER = ROWS_PER_WORKER // GATHER_CHUNK
LANES = 16
PAIR_DIM = 2 * EMBEDDING_DIM


def _positional_encoding():
    pos = np.arange(SEQ_LENGTH, dtype=np.float64)[:, None]
    pe = np.zeros((SEQ_LENGTH, EMBEDDING_DIM), dtype=np.float64)
    i_even = np.arange(0, EMBEDDING_DIM, 2)
    i_odd = i_even + 1
    pe[:, i_even] = np.sin(pos / 10000 ** (2.0 * i_even / EMBEDDING_DIM))
    pe[:, i_odd] = np.cos(pos / 10000 ** (2.0 * i_odd / EMBEDDING_DIM))
    return pe.astype(np.float32)


_PE = _positional_encoding().reshape(
    NUM_WORKERS, CHUNKS_PER_WORKER, GATHER_CHUNK, EMBEDDING_DIM
)


def _sc_embed(table2, idx3, pe4, par4):
    mesh = plsc.VectorSubcoreMesh(core_axis_name="c", subcore_axis_name="s")

    @functools.partial(
        pl.kernel,
        mesh=mesh,
        out_type=jax.ShapeDtypeStruct((SEQ_LENGTH, EMBEDDING_DIM), jnp.float32),
        scratch_types=[
            pltpu.VMEM((CHUNKS_PER_WORKER, GATHER_CHUNK), jnp.int32),
            pltpu.VMEM((CHUNKS_PER_WORKER, GATHER_CHUNK), jnp.int32),
            pltpu.VMEM((GATHER_CHUNK, PAIR_DIM), jnp.float32),
            pltpu.VMEM((GATHER_CHUNK, EMBEDDING_DIM), jnp.float32),
            pltpu.VMEM((GATHER_CHUNK, EMBEDDING_DIM), jnp.int32),
            pltpu.VMEM((GATHER_CHUNK, EMBEDDING_DIM), jnp.float32),
            pltpu.SemaphoreType.DMA,
            pltpu.SemaphoreType.DMA,
            pltpu.SemaphoreType.DMA,
        ],
    )
    def k(
        table_hbm, idx_hbm, pe_hbm, par_hbm, out_hbm,
        idx_v, idx2_v, g_v, pe_v, par_v, o_v, gsem, psem, msem,
    ):
        wid = lax.axis_index("s") * NUM_CORES + lax.axis_index("c")
        base = wid * ROWS_PER_WORKER

        pltpu.sync_copy(idx_hbm.at[wid], idx_v)
        for j in range(CHUNKS_PER_WORKER):
            for c in range(GATHER_CHUNK // LANES):
                sl = pl.ds(c * LANES, LANES)
                idx2_v[j, sl] = idx_v[j, sl] >> 1

        for j in range(CHUNKS_PER_WORKER):
            pe_cp = pltpu.async_copy(pe_hbm.at[wid, j], pe_v, psem)
            par_cp = pltpu.async_copy(par_hbm.at[wid, j], par_v, msem)
            g_cp = pltpu.async_copy(table_hbm.at[idx2_v.at[j]], g_v, gsem)
            g_cp.wait()
            pe_cp.wait()
            par_cp.wait()

            @pl.loop(0, GATHER_CHUNK)
            def _(r):
                for c in range(EMBEDDING_DIM // LANES):
                    sl = pl.ds(c * LANES, LANES)
                    lo = g_v[r, sl]
                    hi = g_v[r, pl.ds(EMBEDDING_DIM + c * LANES, LANES)]
                    sel = jnp.where(par_v[r, sl] != 0, hi, lo)
                    o_v[r, sl] = sel + pe_v[r, sl]

            pltpu.sync_copy(o_v, out_hbm.at[pl.ds(base + j * GATHER_CHUNK, GATHER_CHUNK)])

    return k(table2, idx3, pe4, par4)


def kernel(input_indices, table):
    idx = input_indices.astype(jnp.int32)
    table2 = table.reshape(table.shape[0] // 2, PAIR_DIM)
    idx3 = idx.reshape(NUM_WORKERS, CHUNKS_PER_WORKER, GATHER_CHUNK)
    par4 = jnp.broadcast_to((idx & 1)[:, None], (SEQ_LENGTH, EMBEDDING_DIM)).reshape(
        NUM_WORKERS, CHUNKS_PER_WORKER, GATHER_CHUNK, EMBEDDING_DIM
    )
    pe4 = jnp.asarray(_PE)
    return _sc_embed(table2, idx3, pe4, par4)


# speedup vs baseline: 1.0007x; 1.0007x over previous
import functools

import jax
import jax.numpy as jnp
import numpy as np
from jax import lax
from jax.experimental import pallas as pl
from jax.experimental.pallas import tpu as pltpu
from jax.experimental.pallas import tpu_sc as plsc

SEQ_LENGTH = 8192
EMBEDDING_DIM = 64
NUM_CORES = 2
NUM_SUBCORES = 16
NUM_WORKERS = NUM_CORES * NUM_SUBCORES
ROWS_PER_WORKER = SEQ_LENGTH // NUM_WORKERS
GATHER_CHUNK = 128
CHUNKS_PER_WORKER = ROWS_PER_WORKER // GATHER_CHUNK
LANES = 16
PAIR_DIM = 2 * EMBEDDING_DIM


def _positional_encoding():
    pos = np.arange(SEQ_LENGTH, dtype=np.float64)[:, None]
    pe = np.zeros((SEQ_LENGTH, EMBEDDING_DIM), dtype=np.float64)
    i_even = np.arange(0, EMBEDDING_DIM, 2)
    i_odd = i_even + 1
    pe[:, i_even] = np.sin(pos / 10000 ** (2.0 * i_even / EMBEDDING_DIM))
    pe[:, i_odd] = np.cos(pos / 10000 ** (2.0 * i_odd / EMBEDDING_DIM))
    return pe.astype(np.float32)


_PE = _positional_encoding().reshape(
    NUM_WORKERS, CHUNKS_PER_WORKER, GATHER_CHUNK, EMBEDDING_DIM
)


def _sc_embed(table2, idx3, pe4, par4):
    mesh = plsc.VectorSubcoreMesh(core_axis_name="c", subcore_axis_name="s")

    @functools.partial(
        pl.kernel,
        mesh=mesh,
        compiler_params=pltpu.CompilerParams(use_tc_tiling_on_sc=True),
        out_type=jax.ShapeDtypeStruct((SEQ_LENGTH, EMBEDDING_DIM), jnp.float32),
        scratch_types=[
            pltpu.VMEM((CHUNKS_PER_WORKER, GATHER_CHUNK), jnp.int32),
            pltpu.VMEM((CHUNKS_PER_WORKER, GATHER_CHUNK), jnp.int32),
            pltpu.VMEM((GATHER_CHUNK, PAIR_DIM), jnp.float32),
            pltpu.VMEM((GATHER_CHUNK, EMBEDDING_DIM), jnp.float32),
            pltpu.VMEM((GATHER_CHUNK, EMBEDDING_DIM), jnp.int32),
            pltpu.VMEM((GATHER_CHUNK, EMBEDDING_DIM), jnp.float32),
            pltpu.SemaphoreType.DMA,
            pltpu.SemaphoreType.DMA,
            pltpu.SemaphoreType.DMA,
        ],
    )
    def k(
        table_hbm, idx_hbm, pe_hbm, par_hbm, out_hbm,
        idx_v, idx2_v, g_v, pe_v, par_v, o_v, gsem, psem, msem,
    ):
        wid = lax.axis_index("s") * NUM_CORES + lax.axis_index("c")
        base = wid * ROWS_PER_WORKER

        pltpu.sync_copy(idx_hbm.at[wid], idx_v)
        for j in range(CHUNKS_PER_WORKER):
            for c in range(GATHER_CHUNK // LANES):
                sl = pl.ds(c * LANES, LANES)
                idx2_v[j, sl] = idx_v[j, sl] >> 1

        for j in range(CHUNKS_PER_WORKER):
            pe_cp = pltpu.async_copy(pe_hbm.at[wid, j], pe_v, psem)
            par_cp = pltpu.async_copy(par_hbm.at[wid, j], par_v, msem)
            g_cp = pltpu.async_copy(table_hbm.at[idx2_v.at[j]], g_v, gsem)
            g_cp.wait()
            pe_cp.wait()
            par_cp.wait()

            @pl.loop(0, GATHER_CHUNK)
            def _(r):
                for c in range(EMBEDDING_DIM // LANES):
                    sl = pl.ds(c * LANES, LANES)
                    lo = g_v[r, sl]
                    hi = g_v[r, pl.ds(EMBEDDING_DIM + c * LANES, LANES)]
                    sel = jnp.where(par_v[r, sl] != 0, hi, lo)
                    o_v[r, sl] = sel + pe_v[r, sl]

            pltpu.sync_copy(o_v, out_hbm.at[pl.ds(base + j * GATHER_CHUNK, GATHER_CHUNK)])

    return k(table2, idx3, pe4, par4)


def kernel(input_indices, table):
    idx = input_indices.astype(jnp.int32)
    table2 = table.reshape(table.shape[0] // 2, PAIR_DIM)
    idx3 = idx.reshape(NUM_WORKERS, CHUNKS_PER_WORKER, GATHER_CHUNK)
    par4 = jnp.broadcast_to((idx & 1)[:, None], (SEQ_LENGTH, EMBEDDING_DIM)).reshape(
        NUM_WORKERS, CHUNKS_PER_WORKER, GATHER_CHUNK, EMBEDDING_DIM
    )
    pe4 = jnp.asarray(_PE)
    return _sc_embed(table2, idx3, pe4, par4)


# speedup vs baseline: 3.2422x; 3.2399x over previous
import functools

import jax
import jax.numpy as jnp
import numpy as np
from jax import lax
from jax.experimental import pallas as pl
from jax.experimental.pallas import tpu as pltpu
from jax.experimental.pallas import tpu_sc as plsc

SEQ_LENGTH = 8192
EMBEDDING_DIM = 64
VOCAB = 1000000
NUM_CORES = 2
NUM_SUBCORES = 16
NUM_WORKERS = NUM_CORES * NUM_SUBCORES
ROWS_PER_WORKER = SEQ_LENGTH // NUM_WORKERS
LANES = 16
BLK = 16
NUM_BLKS = ROWS_PER_WORKER // BLK
TAIL_BASE = (VOCAB // 128) * 128
STAGE_W = 128


def _positional_encoding():
    pos = np.arange(SEQ_LENGTH, dtype=np.float64)[:, None]
    pe = np.zeros((SEQ_LENGTH, EMBEDDING_DIM), dtype=np.float64)
    i_even = np.arange(0, EMBEDDING_DIM, 2)
    i_odd = i_even + 1
    pe[:, i_even] = np.sin(pos / 10000 ** (2.0 * i_even / EMBEDDING_DIM))
    pe[:, i_odd] = np.cos(pos / 10000 ** (2.0 * i_odd / EMBEDDING_DIM))
    return pe.astype(np.float32)


_PE = _positional_encoding()


def _sc_gather(table_t, idx2, tail_blk):
    mesh = plsc.VectorSubcoreMesh(core_axis_name="c", subcore_axis_name="s")

    @functools.partial(
        pl.kernel,
        mesh=mesh,
        compiler_params=pltpu.CompilerParams(
            use_tc_tiling_on_sc=True, needs_layout_passes=False
        ),
        out_type=jax.ShapeDtypeStruct((SEQ_LENGTH, 1, STAGE_W), jnp.float32),
        scratch_types=[
            pltpu.VMEM((ROWS_PER_WORKER,), jnp.int32),
            pltpu.VMEM((EMBEDDING_DIM, EMBEDDING_DIM), jnp.float32),
            pltpu.VMEM((2, EMBEDDING_DIM, 128), jnp.float32),
            pltpu.VMEM((BLK, 1, STAGE_W), jnp.float32),
            pltpu.SemaphoreType.DMA,
            pltpu.SemaphoreType.DMA,
            pltpu.SemaphoreType.DMA,
            pltpu.SemaphoreType.DMA,
        ],
    )
    def k(tab_hbm, idx_hbm, tail_hbm, out_hbm,
          idx_v, tail_v, cb_v, row_v, csem0, csem1, tsem, osem):
        wid = lax.axis_index("s") * NUM_CORES + lax.axis_index("c")
        base_row = wid * ROWS_PER_WORKER

        tail_cp = pltpu.async_copy(tail_hbm, tail_v, tsem)
        pltpu.sync_copy(idx_hbm.at[wid], idx_v)
        tail_cp.wait()

        csems = (csem0, csem1)
        cvecs = [
            lax.iota(jnp.int32, LANES) + (b * LANES) for b in range(4)
        ]

        def fire(q, slot):
            cbase = jnp.minimum(q - (q & 127), TAIL_BASE - 128)
            cbase = pl.multiple_of(cbase, 128)
            return pltpu.async_copy(
                tab_hbm.at[:, pl.ds(cbase, 128)], cb_v.at[slot], csems[slot]
            )

        @pl.loop(0, NUM_BLKS)
        def _(b):
            @pl.when(b > 0)
            def _():
                for i in range(BLK):
                    pltpu.make_async_copy(
                        out_hbm.at[0], row_v.at[i], osem
                    ).wait()

            v16 = idx_v[pl.ds(b * BLK, BLK)]
            m16 = v16 & 127
            qs = [v16[i] for i in range(BLK)]
            fire(qs[0], 0)
            for i in range(BLK):
                if i + 1 < BLK:
                    fire(qs[i + 1], (i + 1) % 2)
                pltpu.make_async_copy(
                    tab_hbm.at[:, pl.ds(0, 128)], cb_v.at[i % 2], csems[i % 2]
                ).wait()
                m_splat = jnp.broadcast_to(m16[i], (LANES,))
                is_tail = qs[i] >= TAIL_BASE

                @pl.when(is_tail)
                def _():
                    for b4 in range(4):
                        g = plsc.load_gather(tail_v, [cvecs[b4], m_splat])
                        row_v[i, 0, pl.ds(b4 * LANES, LANES)] = g

                @pl.when(jnp.logical_not(is_tail))
                def _():
                    for b4 in range(4):
                        g = plsc.load_gather(cb_v.at[i % 2], [cvecs[b4], m_splat])
                        row_v[i, 0, pl.ds(b4 * LANES, LANES)] = g

                pltpu.async_copy(
                    row_v.at[i], out_hbm.at[base_row + b * BLK + i], osem
                )

        for i in range(BLK):
            pltpu.make_async_copy(out_hbm.at[0], row_v.at[i], osem).wait()

    return k(table_t, idx2, tail_blk)


def _tc_add_pe(staged, pe):
    grid = 16
    rows = SEQ_LENGTH // grid

    def body(g_ref, pe_ref, o_ref):
        o_ref[...] = g_ref[:, :EMBEDDING_DIM] + pe_ref[...]

    return pl.pallas_call(
        body,
        grid=(grid,),
        in_specs=[
            pl.BlockSpec((rows, STAGE_W), lambda i: (i, 0)),
            pl.BlockSpec((rows, EMBEDDING_DIM), lambda i: (i, 0)),
        ],
        out_specs=pl.BlockSpec((rows, EMBEDDING_DIM), lambda i: (i, 0)),
        out_shape=jax.ShapeDtypeStruct((SEQ_LENGTH, EMBEDDING_DIM), jnp.float32),
    )(staged, pe)


def kernel(input_indices, table):
    idx2 = input_indices.astype(jnp.int32).reshape(NUM_WORKERS, ROWS_PER_WORKER)
    table_t = table.T
    tail_blk = lax.slice(table_t, (0, TAIL_BASE), (EMBEDDING_DIM, VOCAB))
    staged = _sc_gather(table_t, idx2, tail_blk)
    pe = jnp.asarray(_PE)
    return _tc_add_pe(staged.reshape(SEQ_LENGTH, STAGE_W), pe)


# speedup vs baseline: 4.1014x; 1.2650x over previous
import functools

import jax
import jax.numpy as jnp
import numpy as np
from jax import lax
from jax.experimental import pallas as pl
from jax.experimental.pallas import tpu as pltpu
from jax.experimental.pallas import tpu_sc as plsc

SEQ_LENGTH = 8192
EMBEDDING_DIM = 64
VOCAB = 1000000
NUM_CORES = 2
NUM_SUBCORES = 16
NUM_WORKERS = NUM_CORES * NUM_SUBCORES
ROWS_PER_WORKER = SEQ_LENGTH // NUM_WORKERS
LANES = 16
BLK = 32
NUM_BLKS = ROWS_PER_WORKER // BLK
NSLOTS = 4
TAIL_BASE = (VOCAB // 128) * 128
STAGE_W = 128


def _positional_encoding():
    pos = np.arange(SEQ_LENGTH, dtype=np.float64)[:, None]
    pe = np.zeros((SEQ_LENGTH, EMBEDDING_DIM), dtype=np.float64)
    i_even = np.arange(0, EMBEDDING_DIM, 2)
    i_odd = i_even + 1
    pe[:, i_even] = np.sin(pos / 10000 ** (2.0 * i_even / EMBEDDING_DIM))
    pe[:, i_odd] = np.cos(pos / 10000 ** (2.0 * i_odd / EMBEDDING_DIM))
    return pe.astype(np.float32)


_PE = _positional_encoding()


def _sc_gather(table_t, idx2, tail_blk):
    mesh = plsc.VectorSubcoreMesh(core_axis_name="c", subcore_axis_name="s")

    @functools.partial(
        pl.kernel,
        mesh=mesh,
        compiler_params=pltpu.CompilerParams(
            use_tc_tiling_on_sc=True, needs_layout_passes=False
        ),
        out_type=jax.ShapeDtypeStruct((SEQ_LENGTH, 1, STAGE_W), jnp.float32),
        scratch_types=[
            pltpu.VMEM((ROWS_PER_WORKER,), jnp.int32),
            pltpu.VMEM((EMBEDDING_DIM, EMBEDDING_DIM), jnp.float32),
            pltpu.VMEM((NSLOTS, EMBEDDING_DIM, 128), jnp.float32),
            pltpu.VMEM((BLK, 1, STAGE_W), jnp.float32),
            pltpu.SemaphoreType.DMA,
            pltpu.SemaphoreType.DMA,
            pltpu.SemaphoreType.DMA,
            pltpu.SemaphoreType.DMA,
            pltpu.SemaphoreType.DMA,
            pltpu.SemaphoreType.DMA,
        ],
    )
    def k(tab_hbm, idx_hbm, tail_hbm, out_hbm,
          idx_v, tail_v, cb_v, row_v, csem0, csem1, csem2, csem3, tsem, osem):
        wid = lax.axis_index("s") * NUM_CORES + lax.axis_index("c")
        base_row = wid * ROWS_PER_WORKER

        tail_cp = pltpu.async_copy(tail_hbm, tail_v, tsem)
        pltpu.sync_copy(idx_hbm.at[wid], idx_v)
        tail_cp.wait()

        csems = (csem0, csem1, csem2, csem3)
        cvecs = [
            lax.iota(jnp.int32, LANES) + (b * LANES) for b in range(4)
        ]

        def fire(q, slot):
            cbase = jnp.minimum(q - (q & 127), TAIL_BASE - 128)
            cbase = pl.multiple_of(cbase, 128)
            return pltpu.async_copy(
                tab_hbm.at[:, pl.ds(cbase, 128)], cb_v.at[slot], csems[slot]
            )

        @pl.loop(0, NUM_BLKS)
        def _(b):
            @pl.when(b > 0)
            def _():
                for i in range(BLK):
                    pltpu.make_async_copy(
                        out_hbm.at[0], row_v.at[i], osem
                    ).wait()

            vs = [idx_v[pl.ds((b * BLK) + h * LANES, LANES)] for h in range(BLK // LANES)]
            ms = [v & 127 for v in vs]
            qs = [v[i] for v in vs for i in range(LANES)]
            for j in range(NSLOTS - 1):
                fire(qs[j], j)
            for i in range(BLK):
                if i + NSLOTS - 1 < BLK:
                    fire(qs[i + NSLOTS - 1], (i + NSLOTS - 1) % NSLOTS)
                slot = i % NSLOTS
                pltpu.make_async_copy(
                    tab_hbm.at[:, pl.ds(0, 128)], cb_v.at[slot], csems[slot]
                ).wait()
                m_splat = jnp.broadcast_to(ms[i // LANES][i % LANES], (LANES,))
                is_tail = qs[i] >= TAIL_BASE

                @pl.when(is_tail)
                def _():
                    for b4 in range(4):
                        g = plsc.load_gather(tail_v, [cvecs[b4], m_splat])
                        row_v[i, 0, pl.ds(b4 * LANES, LANES)] = g

                @pl.when(jnp.logical_not(is_tail))
                def _():
                    for b4 in range(4):
                        g = plsc.load_gather(cb_v.at[slot], [cvecs[b4], m_splat])
                        row_v[i, 0, pl.ds(b4 * LANES, LANES)] = g

                pltpu.async_copy(
                    row_v.at[i], out_hbm.at[base_row + b * BLK + i], osem
                )

        for i in range(BLK):
            pltpu.make_async_copy(out_hbm.at[0], row_v.at[i], osem).wait()

    return k(table_t, idx2, tail_blk)


def _tc_add_pe(staged, pe):
    grid = 16
    rows = SEQ_LENGTH // grid

    def body(g_ref, pe_ref, o_ref):
        o_ref[...] = g_ref[:, :EMBEDDING_DIM] + pe_ref[...]

    return pl.pallas_call(
        body,
        grid=(grid,),
        in_specs=[
            pl.BlockSpec((rows, STAGE_W), lambda i: (i, 0)),
            pl.BlockSpec((rows, EMBEDDING_DIM), lambda i: (i, 0)),
        ],
        out_specs=pl.BlockSpec((rows, EMBEDDING_DIM), lambda i: (i, 0)),
        out_shape=jax.ShapeDtypeStruct((SEQ_LENGTH, EMBEDDING_DIM), jnp.float32),
    )(staged, pe)


def kernel(input_indices, table):
    idx2 = input_indices.astype(jnp.int32).reshape(NUM_WORKERS, ROWS_PER_WORKER)
    table_t = table.T
    tail_blk = lax.slice(table_t, (0, TAIL_BASE), (EMBEDDING_DIM, VOCAB))
    staged = _sc_gather(table_t, idx2, tail_blk)
    pe = jnp.asarray(_PE)
    return _tc_add_pe(staged.reshape(SEQ_LENGTH, STAGE_W), pe)


# speedup vs baseline: 4.4116x; 1.0756x over previous
import functools

import jax
import jax.numpy as jnp
import numpy as np
from jax import lax
from jax.experimental import pallas as pl
from jax.experimental.pallas import tpu as pltpu
from jax.experimental.pallas import tpu_sc as plsc

SEQ_LENGTH = 8192
EMBEDDING_DIM = 64
VOCAB = 1000000
NUM_CORES = 2
NUM_SUBCORES = 16
NUM_WORKERS = NUM_CORES * NUM_SUBCORES
ROWS_PER_WORKER = SEQ_LENGTH // NUM_WORKERS
LANES = 16
BLK = 32
NUM_BLKS = ROWS_PER_WORKER // BLK
NSLOTS = 6
TAIL_BASE = (VOCAB // 128) * 128
STAGE_W = 128


def _positional_encoding():
    pos = np.arange(SEQ_LENGTH, dtype=np.float64)[:, None]
    pe = np.zeros((SEQ_LENGTH, EMBEDDING_DIM), dtype=np.float64)
    i_even = np.arange(0, EMBEDDING_DIM, 2)
    i_odd = i_even + 1
    pe[:, i_even] = np.sin(pos / 10000 ** (2.0 * i_even / EMBEDDING_DIM))
    pe[:, i_odd] = np.cos(pos / 10000 ** (2.0 * i_odd / EMBEDDING_DIM))
    return pe.astype(np.float32)


_PE = _positional_encoding()


def _sc_gather(table_t, idx2, tail_blk):
    mesh = plsc.VectorSubcoreMesh(core_axis_name="c", subcore_axis_name="s")

    @functools.partial(
        pl.kernel,
        mesh=mesh,
        compiler_params=pltpu.CompilerParams(
            use_tc_tiling_on_sc=True, needs_layout_passes=False
        ),
        out_type=jax.ShapeDtypeStruct((SEQ_LENGTH, 1, STAGE_W), jnp.float32),
        scratch_types=[
            pltpu.VMEM((ROWS_PER_WORKER,), jnp.int32),
            pltpu.VMEM((EMBEDDING_DIM, EMBEDDING_DIM), jnp.float32),
            pltpu.VMEM((NSLOTS, EMBEDDING_DIM, 128), jnp.float32),
            pltpu.VMEM((BLK, 1, STAGE_W), jnp.float32),
            pltpu.SemaphoreType.DMA,
            pltpu.SemaphoreType.DMA,
            pltpu.SemaphoreType.DMA,
            pltpu.SemaphoreType.DMA,
            pltpu.SemaphoreType.DMA,
            pltpu.SemaphoreType.DMA,
            pltpu.SemaphoreType.DMA,
            pltpu.SemaphoreType.DMA,
        ],
    )
    def k(tab_hbm, idx_hbm, tail_hbm, out_hbm,
          idx_v, tail_v, cb_v, row_v,
          csem0, csem1, csem2, csem3, csem4, csem5, tsem, osem):
        wid = lax.axis_index("s") * NUM_CORES + lax.axis_index("c")
        base_row = wid * ROWS_PER_WORKER

        tail_cp = pltpu.async_copy(tail_hbm, tail_v, tsem)
        pltpu.sync_copy(idx_hbm.at[wid], idx_v)
        tail_cp.wait()

        csems = (csem0, csem1, csem2, csem3, csem4, csem5)
        cvecs = [
            lax.iota(jnp.int32, LANES) + (b * LANES) for b in range(4)
        ]

        def fire(q, slot):
            cbase = jnp.minimum(q - (q & 127), TAIL_BASE - 128)
            cbase = pl.multiple_of(cbase, 128)
            return pltpu.async_copy(
                tab_hbm.at[:, pl.ds(cbase, 128)], cb_v.at[slot], csems[slot]
            )

        @pl.loop(0, NUM_BLKS)
        def _(b):
            @pl.when(b > 0)
            def _():
                for i in range(BLK):
                    pltpu.make_async_copy(
                        out_hbm.at[0], row_v.at[i], osem
                    ).wait()

            vs = [idx_v[pl.ds((b * BLK) + h * LANES, LANES)] for h in range(BLK // LANES)]
            ms = [v & 127 for v in vs]
            qs = [v[i] for v in vs for i in range(LANES)]
            for j in range(NSLOTS - 1):
                fire(qs[j], j)
            for i in range(BLK):
                if i + NSLOTS - 1 < BLK:
                    fire(qs[i + NSLOTS - 1], (i + NSLOTS - 1) % NSLOTS)
                slot = i % NSLOTS
                pltpu.make_async_copy(
                    tab_hbm.at[:, pl.ds(0, 128)], cb_v.at[slot], csems[slot]
                ).wait()
                m_splat = jnp.broadcast_to(ms[i // LANES][i % LANES], (LANES,))
                is_tail = qs[i] >= TAIL_BASE

                @pl.when(is_tail)
                def _():
                    for b4 in range(4):
                        g = plsc.load_gather(tail_v, [cvecs[b4], m_splat])
                        row_v[i, 0, pl.ds(b4 * LANES, LANES)] = g

                @pl.when(jnp.logical_not(is_tail))
                def _():
                    for b4 in range(4):
                        g = plsc.load_gather(cb_v.at[slot], [cvecs[b4], m_splat])
                        row_v[i, 0, pl.ds(b4 * LANES, LANES)] = g

                pltpu.async_copy(
                    row_v.at[i], out_hbm.at[base_row + b * BLK + i], osem
                )

        for i in range(BLK):
            pltpu.make_async_copy(out_hbm.at[0], row_v.at[i], osem).wait()

    return k(table_t, idx2, tail_blk)


def _tc_add_pe(staged, pe):
    grid = 16
    rows = SEQ_LENGTH // grid

    def body(g_ref, pe_ref, o_ref):
        o_ref[...] = g_ref[:, :EMBEDDING_DIM] + pe_ref[...]

    return pl.pallas_call(
        body,
        grid=(grid,),
        in_specs=[
            pl.BlockSpec((rows, STAGE_W), lambda i: (i, 0)),
            pl.BlockSpec((rows, EMBEDDING_DIM), lambda i: (i, 0)),
        ],
        out_specs=pl.BlockSpec((rows, EMBEDDING_DIM), lambda i: (i, 0)),
        out_shape=jax.ShapeDtypeStruct((SEQ_LENGTH, EMBEDDING_DIM), jnp.float32),
    )(staged, pe)


def kernel(input_indices, table):
    idx2 = input_indices.astype(jnp.int32).reshape(NUM_WORKERS, ROWS_PER_WORKER)
    table_t = table.T
    tail_blk = lax.slice(table_t, (0, TAIL_BASE), (EMBEDDING_DIM, VOCAB))
    staged = _sc_gather(table_t, idx2, tail_blk)
    pe = jnp.asarray(_PE)
    return _tc_add_pe(staged.reshape(SEQ_LENGTH, STAGE_W), pe)


# speedup vs baseline: 4.5795x; 1.0380x over previous
import functools

import jax
import jax.numpy as jnp
import numpy as np
from jax import lax
from jax.experimental import pallas as pl
from jax.experimental.pallas import tpu as pltpu
from jax.experimental.pallas import tpu_sc as plsc

SEQ_LENGTH = 8192
EMBEDDING_DIM = 64
VOCAB = 1000000
NUM_CORES = 2
NUM_SUBCORES = 16
NUM_WORKERS = NUM_CORES * NUM_SUBCORES
ROWS_PER_WORKER = SEQ_LENGTH // NUM_WORKERS
LANES = 16
BLK = 64
NUM_BLKS = ROWS_PER_WORKER // BLK
NSLOTS = 6
TAIL_BASE = (VOCAB // 128) * 128
STAGE_W = 128


def _positional_encoding():
    pos = np.arange(SEQ_LENGTH, dtype=np.float64)[:, None]
    pe = np.zeros((SEQ_LENGTH, EMBEDDING_DIM), dtype=np.float64)
    i_even = np.arange(0, EMBEDDING_DIM, 2)
    i_odd = i_even + 1
    pe[:, i_even] = np.sin(pos / 10000 ** (2.0 * i_even / EMBEDDING_DIM))
    pe[:, i_odd] = np.cos(pos / 10000 ** (2.0 * i_odd / EMBEDDING_DIM))
    return pe.astype(np.float32)


_PE = _positional_encoding()


def _sc_gather(table_t, idx2, tail_blk):
    mesh = plsc.VectorSubcoreMesh(core_axis_name="c", subcore_axis_name="s")

    @functools.partial(
        pl.kernel,
        mesh=mesh,
        compiler_params=pltpu.CompilerParams(
            use_tc_tiling_on_sc=True, needs_layout_passes=False
        ),
        out_type=jax.ShapeDtypeStruct((SEQ_LENGTH, 1, STAGE_W), jnp.float32),
        scratch_types=[
            pltpu.VMEM((ROWS_PER_WORKER,), jnp.int32),
            pltpu.VMEM((EMBEDDING_DIM, EMBEDDING_DIM), jnp.float32),
            pltpu.VMEM((NSLOTS, EMBEDDING_DIM, 128), jnp.float32),
            pltpu.VMEM((BLK, 1, STAGE_W), jnp.float32),
            pltpu.SemaphoreType.DMA,
            pltpu.SemaphoreType.DMA,
            pltpu.SemaphoreType.DMA,
            pltpu.SemaphoreType.DMA,
            pltpu.SemaphoreType.DMA,
            pltpu.SemaphoreType.DMA,
            pltpu.SemaphoreType.DMA,
            pltpu.SemaphoreType.DMA,
        ],
    )
    def k(tab_hbm, idx_hbm, tail_hbm, out_hbm,
          idx_v, tail_v, cb_v, row_v,
          csem0, csem1, csem2, csem3, csem4, csem5, tsem, osem):
        wid = lax.axis_index("s") * NUM_CORES + lax.axis_index("c")
        base_row = wid * ROWS_PER_WORKER

        tail_cp = pltpu.async_copy(tail_hbm, tail_v, tsem)
        pltpu.sync_copy(idx_hbm.at[wid], idx_v)
        tail_cp.wait()

        csems = (csem0, csem1, csem2, csem3, csem4, csem5)
        cvecs = [
            lax.iota(jnp.int32, LANES) + (b * LANES) for b in range(4)
        ]

        def fire(q, slot):
            cbase = jnp.minimum(q - (q & 127), TAIL_BASE - 128)
            cbase = pl.multiple_of(cbase, 128)
            return pltpu.async_copy(
                tab_hbm.at[:, pl.ds(cbase, 128)], cb_v.at[slot], csems[slot]
            )

        @pl.loop(0, NUM_BLKS)
        def _(b):
            @pl.when(b > 0)
            def _():
                for i in range(BLK):
                    pltpu.make_async_copy(
                        out_hbm.at[0], row_v.at[i], osem
                    ).wait()

            vs = [idx_v[pl.ds((b * BLK) + h * LANES, LANES)] for h in range(BLK // LANES)]
            ms = [v & 127 for v in vs]
            qs = [v[i] for v in vs for i in range(LANES)]
            for j in range(NSLOTS - 1):
                fire(qs[j], j)
            for i in range(BLK):
                if i + NSLOTS - 1 < BLK:
                    fire(qs[i + NSLOTS - 1], (i + NSLOTS - 1) % NSLOTS)
                slot = i % NSLOTS
                pltpu.make_async_copy(
                    tab_hbm.at[:, pl.ds(0, 128)], cb_v.at[slot], csems[slot]
                ).wait()
                m_splat = jnp.broadcast_to(ms[i // LANES][i % LANES], (LANES,))
                is_tail = qs[i] >= TAIL_BASE

                @pl.when(is_tail)
                def _():
                    for b4 in range(4):
                        g = plsc.load_gather(tail_v, [cvecs[b4], m_splat])
                        row_v[i, 0, pl.ds(b4 * LANES, LANES)] = g

                @pl.when(jnp.logical_not(is_tail))
                def _():
                    for b4 in range(4):
                        g = plsc.load_gather(cb_v.at[slot], [cvecs[b4], m_splat])
                        row_v[i, 0, pl.ds(b4 * LANES, LANES)] = g

                pltpu.async_copy(
                    row_v.at[i], out_hbm.at[base_row + b * BLK + i], osem
                )

        for i in range(BLK):
            pltpu.make_async_copy(out_hbm.at[0], row_v.at[i], osem).wait()

    return k(table_t, idx2, tail_blk)


def _tc_add_pe(staged, pe):
    grid = 16
    rows = SEQ_LENGTH // grid

    def body(g_ref, pe_ref, o_ref):
        o_ref[...] = (g_ref[:, :EMBEDDING_DIM] + pe_ref[...]).T

    return pl.pallas_call(
        body,
        grid=(grid,),
        in_specs=[
            pl.BlockSpec((rows, STAGE_W), lambda i: (i, 0)),
            pl.BlockSpec((rows, EMBEDDING_DIM), lambda i: (i, 0)),
        ],
        out_specs=pl.BlockSpec((EMBEDDING_DIM, rows), lambda i: (0, i)),
        out_shape=jax.ShapeDtypeStruct((EMBEDDING_DIM, SEQ_LENGTH), jnp.float32),
    )(staged, pe)


def kernel(input_indices, table):
    idx2 = input_indices.astype(jnp.int32).reshape(NUM_WORKERS, ROWS_PER_WORKER)
    table_t = table.T
    tail_blk = lax.slice(table_t, (0, TAIL_BASE), (EMBEDDING_DIM, VOCAB))
    staged = _sc_gather(table_t, idx2, tail_blk)
    pe = jnp.asarray(_PE)
    return _tc_add_pe(staged.reshape(SEQ_LENGTH, STAGE_W), pe).T


# speedup vs baseline: 4.5983x; 1.0041x over previous
import functools

import jax
import jax.numpy as jnp
import numpy as np
from jax import lax
from jax.experimental import pallas as pl
from jax.experimental.pallas import tpu as pltpu
from jax.experimental.pallas import tpu_sc as plsc

SEQ_LENGTH = 8192
EMBEDDING_DIM = 64
VOCAB = 1000000
NUM_CORES = 2
NUM_SUBCORES = 16
NUM_WORKERS = NUM_CORES * NUM_SUBCORES
ROWS_PER_WORKER = SEQ_LENGTH // NUM_WORKERS
LANES = 16
BLK = 64
NUM_BLKS = ROWS_PER_WORKER // BLK
NSLOTS = 6
TAIL_BASE = (VOCAB // 128) * 128
STAGE_W = 128


def _positional_encoding():
    pos = np.arange(SEQ_LENGTH, dtype=np.float64)[:, None]
    pe = np.zeros((SEQ_LENGTH, EMBEDDING_DIM), dtype=np.float64)
    i_even = np.arange(0, EMBEDDING_DIM, 2)
    i_odd = i_even + 1
    pe[:, i_even] = np.sin(pos / 10000 ** (2.0 * i_even / EMBEDDING_DIM))
    pe[:, i_odd] = np.cos(pos / 10000 ** (2.0 * i_odd / EMBEDDING_DIM))
    return pe.astype(np.float32)


_PE = _positional_encoding()


def _sc_gather(table_t, idx2, tail_blk):
    mesh = plsc.VectorSubcoreMesh(core_axis_name="c", subcore_axis_name="s")

    @functools.partial(
        pl.kernel,
        mesh=mesh,
        compiler_params=pltpu.CompilerParams(
            use_tc_tiling_on_sc=True, needs_layout_passes=False
        ),
        out_type=jax.ShapeDtypeStruct((SEQ_LENGTH, 1, STAGE_W), jnp.float32),
        scratch_types=[
            pltpu.VMEM((ROWS_PER_WORKER,), jnp.int32),
            pltpu.VMEM((EMBEDDING_DIM, EMBEDDING_DIM), jnp.float32),
            pltpu.VMEM((NSLOTS, EMBEDDING_DIM, 128), jnp.float32),
            pltpu.VMEM((BLK, 1, STAGE_W), jnp.float32),
            pltpu.SemaphoreType.DMA,
            pltpu.SemaphoreType.DMA,
            pltpu.SemaphoreType.DMA,
            pltpu.SemaphoreType.DMA,
            pltpu.SemaphoreType.DMA,
            pltpu.SemaphoreType.DMA,
            pltpu.SemaphoreType.DMA,
            pltpu.SemaphoreType.DMA,
        ],
    )
    def k(tab_hbm, idx_hbm, tail_hbm, out_hbm,
          idx_v, tail_v, cb_v, row_v,
          csem0, csem1, csem2, csem3, csem4, csem5, tsem, osem):
        wid = lax.axis_index("s") * NUM_CORES + lax.axis_index("c")
        base_row = wid * ROWS_PER_WORKER

        tail_cp = pltpu.async_copy(tail_hbm, tail_v, tsem)
        pltpu.sync_copy(idx_hbm.at[wid], idx_v)
        tail_cp.wait()

        csems = (csem0, csem1, csem2, csem3, csem4, csem5)
        cvecs = [
            lax.iota(jnp.int32, LANES) + (b * LANES) for b in range(4)
        ]

        def fire(q, slot):
            cbase = jnp.minimum(q - (q & 127), TAIL_BASE - 128)
            cbase = pl.multiple_of(cbase, 128)
            return pltpu.async_copy(
                tab_hbm.at[:, pl.ds(cbase, 128)], cb_v.at[slot], csems[slot]
            )

        @pl.loop(0, NUM_BLKS)
        def _(b):
            @pl.when(b > 0)
            def _():
                pltpu.make_async_copy(
                    out_hbm.at[pl.ds(0, BLK)], row_v, osem
                ).wait()

            vs = [idx_v[pl.ds((b * BLK) + h * LANES, LANES)] for h in range(BLK // LANES)]
            ms = [v & 127 for v in vs]
            qs = [v[i] for v in vs for i in range(LANES)]
            for j in range(NSLOTS - 1):
                fire(qs[j], j)
            for i in range(BLK):
                if i + NSLOTS - 1 < BLK:
                    fire(qs[i + NSLOTS - 1], (i + NSLOTS - 1) % NSLOTS)
                slot = i % NSLOTS
                pltpu.make_async_copy(
                    tab_hbm.at[:, pl.ds(0, 128)], cb_v.at[slot], csems[slot]
                ).wait()
                m_splat = jnp.broadcast_to(ms[i // LANES][i % LANES], (LANES,))
                is_tail = qs[i] >= TAIL_BASE

                @pl.when(is_tail)
                def _():
                    for b4 in range(4):
                        g = plsc.load_gather(tail_v, [cvecs[b4], m_splat])
                        row_v[i, 0, pl.ds(b4 * LANES, LANES)] = g

                @pl.when(jnp.logical_not(is_tail))
                def _():
                    for b4 in range(4):
                        g = plsc.load_gather(cb_v.at[slot], [cvecs[b4], m_splat])
                        row_v[i, 0, pl.ds(b4 * LANES, LANES)] = g

            pltpu.async_copy(
                row_v, out_hbm.at[pl.ds(base_row + b * BLK, BLK)], osem
            )

        pltpu.make_async_copy(out_hbm.at[pl.ds(0, BLK)], row_v, osem).wait()

    return k(table_t, idx2, tail_blk)


def _tc_add_pe(staged, pe):
    grid = 16
    rows = SEQ_LENGTH // grid

    def body(g_ref, pe_ref, o_ref):
        o_ref[...] = (g_ref[:, :EMBEDDING_DIM] + pe_ref[...]).T

    return pl.pallas_call(
        body,
        grid=(grid,),
        in_specs=[
            pl.BlockSpec((rows, STAGE_W), lambda i: (i, 0)),
            pl.BlockSpec((rows, EMBEDDING_DIM), lambda i: (i, 0)),
        ],
        out_specs=pl.BlockSpec((EMBEDDING_DIM, rows), lambda i: (0, i)),
        out_shape=jax.ShapeDtypeStruct((EMBEDDING_DIM, SEQ_LENGTH), jnp.float32),
    )(staged, pe)


def kernel(input_indices, table):
    idx2 = input_indices.astype(jnp.int32).reshape(NUM_WORKERS, ROWS_PER_WORKER)
    table_t = table.T
    tail_blk = lax.slice(table_t, (0, TAIL_BASE), (EMBEDDING_DIM, VOCAB))
    staged = _sc_gather(table_t, idx2, tail_blk)
    pe = jnp.asarray(_PE)
    return _tc_add_pe(staged.reshape(SEQ_LENGTH, STAGE_W), pe).T
